# Initial kernel scaffold; baseline (speedup 1.0000x reference)
import jax
import jax.numpy as jnp
from jax.experimental import pallas as pl


def kernel(x, edge_feat, edge_index, W1, b1, W2, b2):
    raise NotImplementedError("write your pallas kernel here")


# speedup vs baseline: 2.4439x; 2.4439x over previous
import functools

import jax
import jax.numpy as jnp
from jax import lax
from jax.experimental import pallas as pl
from jax.experimental.pallas import tpu as pltpu
from jax.experimental.pallas import tpu_sc as plsc

NC = 2
NS = 16
NW = NC * NS
K = 64
LANES = 16
CW = 128


def _xw_bias_kernel(x_ref, w_ref, b_ref, o_ref):
    o_ref[...] = (
        jnp.dot(x_ref[...], w_ref[...], preferred_element_type=jnp.float32)
        + b_ref[...]
    )


def _ew_kernel(e_ref, w_ref, o_ref):
    o_ref[...] = jnp.dot(e_ref[...], w_ref[...],
                         preferred_element_type=jnp.float32)


def _node_update_kernel(sum_ref, cnt_ref, x_ref, wa_ref, wb_ref, b_ref, o_ref):
    s = sum_ref[0] + sum_ref[1]
    cnt = (cnt_ref[0] + cnt_ref[1])[:, 0:1]
    red = s / jnp.maximum(cnt, 1.0)
    o_ref[...] = jnp.maximum(
        jnp.dot(red, wa_ref[...], preferred_element_type=jnp.float32)
        + jnp.dot(x_ref[...], wb_ref[...], preferred_element_type=jnp.float32)
        + b_ref[...],
        0.0,
    )


def _sc_edge_sum(ch, np_rows, d,
                 xe, ef1, srci, dsti, out_sum,
                 vsrc, vdst, vrows, vef, acc, sem):
    c = lax.axis_index("c")
    s = lax.axis_index("s")
    wid = s * NC + c
    ng = d // LANES
    z16 = jnp.zeros((LANES,), jnp.float32)

    @pl.loop(0, K)
    def _zero(i):
        for g in range(ng):
            vrows[i, pl.ds(g * LANES, LANES)] = z16

    rows_per_tile = np_rows // NS
    nslab = rows_per_tile // K
    for k in range(nslab):
        sl = pl.ds(s * rows_per_tile + k * K, K)
        pltpu.sync_copy(vrows, acc.at[sl])

    plsc.subcore_barrier()

    @pl.loop(0, ch)
    def _chunk(j):
        pltpu.sync_copy(srci.at[wid, j], vsrc.at[0])
        pltpu.sync_copy(dsti.at[wid, j], vdst.at[0])
        pltpu.async_copy(xe.at[vsrc.at[0]], vrows, sem).wait()
        pltpu.sync_copy(ef1.at[pl.ds((wid * ch + j) * K, K)], vef)

        @pl.loop(0, K)
        def _row(i):
            for g in range(ng):
                sl = pl.ds(g * LANES, LANES)
                vrows[i, sl] = jnp.maximum(vrows[i, sl] + vef[i, sl], 0.0)

        pltpu.sync_copy(vrows, acc.at[vdst.at[0]], add=True)

    plsc.subcore_barrier()

    for k in range(nslab):
        sl = pl.ds(s * rows_per_tile + k * K, K)
        pltpu.sync_copy(acc.at[sl], vrows)
        pltpu.sync_copy(vrows, out_sum.at[c, sl])


def _sc_edge_cnt(ch, np_rows,
                 dsti, out_cnt, vdst, vones, acc):
    c = lax.axis_index("c")
    s = lax.axis_index("s")
    wid = s * NC + c
    z16 = jnp.zeros((LANES,), jnp.float32)
    one0 = jnp.where(lax.iota(jnp.int32, LANES) == 0,
                     jnp.float32(1.0), jnp.float32(0.0))
    ng = CW // LANES

    @pl.loop(0, K)
    def _zero(i):
        vones[i, pl.ds(0, LANES)] = one0
        for g in range(1, ng):
            vones[i, pl.ds(g * LANES, LANES)] = z16

    rows_per_tile = np_rows // NS
    nslab = rows_per_tile // K

    @pl.loop(0, K)
    def _zero2(i):
        vones[i, pl.ds(0, LANES)] = z16
    for k in range(nslab):
        sl = pl.ds(s * rows_per_tile + k * K, K)
        pltpu.sync_copy(vones, acc.at[sl])

    @pl.loop(0, K)
    def _ones(i):
        vones[i, pl.ds(0, LANES)] = one0

    plsc.subcore_barrier()

    @pl.loop(0, ch)
    def _chunk(j):
        pltpu.sync_copy(dsti.at[wid, j], vdst.at[0])
        pltpu.sync_copy(vones, acc.at[vdst.at[0]], add=True)

    plsc.subcore_barrier()

    for k in range(nslab):
        sl = pl.ds(s * rows_per_tile + k * K, K)
        pltpu.sync_copy(acc.at[sl], vones)
        pltpu.sync_copy(vones, out_cnt.at[c, sl])


def kernel(x, edge_feat, edge_index, W1, b1, W2, b2):
    N, D = x.shape
    E, DE = edge_feat.shape

    W1e, W1x = W1[:DE], W1[DE:]
    W2a, W2b = W2[:D], W2[D:]
    b1r = b1.reshape(1, D)
    b2r = b2.reshape(1, D)

    CH = -(-E // (NW * K))
    EP = NW * CH * K
    pad_e = EP - E
    src = edge_index[0]
    dst = edge_index[1]
    srcp = jnp.pad(src, (0, pad_e)).reshape(NW, CH, K)
    dstp = jnp.pad(dst, (0, pad_e), constant_values=N).reshape(NW, CH, K)
    efp = jnp.pad(edge_feat, ((0, pad_e), (0, 0)))

    NP = -(-(N + 1) // (NS * K)) * (NS * K)
    BN = 512
    NP2 = -(-N // BN) * BN
    xp = jnp.pad(x, ((0, NP2 - N), (0, 0)))

    xe = pl.pallas_call(
        _xw_bias_kernel,
        grid=(NP2 // BN,),
        in_specs=[pl.BlockSpec((BN, D), lambda i: (i, 0)),
                  pl.BlockSpec((D, D), lambda i: (0, 0)),
                  pl.BlockSpec((1, D), lambda i: (0, 0))],
        out_specs=pl.BlockSpec((BN, D), lambda i: (i, 0)),
        out_shape=jax.ShapeDtypeStruct((NP2, D), jnp.float32),
    )(xp, W1x, b1r)

    BE = NW * K
    ef1 = pl.pallas_call(
        _ew_kernel,
        grid=(EP // BE,),
        in_specs=[pl.BlockSpec((BE, DE), lambda i: (i, 0)),
                  pl.BlockSpec((DE, D), lambda i: (0, 0))],
        out_specs=pl.BlockSpec((BE, D), lambda i: (i, 0)),
        out_shape=jax.ShapeDtypeStruct((EP, D), jnp.float32),
    )(efp, W1e)

    mesh = plsc.VectorSubcoreMesh(core_axis_name="c", subcore_axis_name="s")
    sc_sum = pl.kernel(
        functools.partial(_sc_edge_sum, CH, NP, D),
        out_type=jax.ShapeDtypeStruct((NC, NP, D), jnp.float32),
        mesh=mesh,
        scratch_types=[
            pltpu.VMEM((1, K), jnp.int32),
            pltpu.VMEM((1, K), jnp.int32),
            pltpu.VMEM((K, D), jnp.float32),
            pltpu.VMEM((K, D), jnp.float32),
            pltpu.VMEM_SHARED((NP, D), jnp.float32),
            pltpu.SemaphoreType.DMA,
        ],
    )
    sums = sc_sum(xe, ef1, srcp, dstp)

    sc_cnt = pl.kernel(
        functools.partial(_sc_edge_cnt, CH, NP),
        out_type=jax.ShapeDtypeStruct((NC, NP, CW), jnp.float32),
        mesh=mesh,
        scratch_types=[
            pltpu.VMEM((1, K), jnp.int32),
            pltpu.VMEM((K, CW), jnp.float32),
            pltpu.VMEM_SHARED((NP, CW), jnp.float32),
        ],
    )
    cnts = sc_cnt(dstp)

    hid = pl.pallas_call(
        _node_update_kernel,
        grid=(NP2 // BN,),
        in_specs=[pl.BlockSpec((NC, BN, D), lambda i: (0, i, 0)),
                  pl.BlockSpec((NC, BN, CW), lambda i: (0, i, 0)),
                  pl.BlockSpec((BN, D), lambda i: (i, 0)),
                  pl.BlockSpec((D, D), lambda i: (0, 0)),
                  pl.BlockSpec((D, D), lambda i: (0, 0)),
                  pl.BlockSpec((1, D), lambda i: (0, 0))],
        out_specs=pl.BlockSpec((BN, D), lambda i: (i, 0)),
        out_shape=jax.ShapeDtypeStruct((NP2, D), jnp.float32),
    )(sums, cnts, xp, W2a, W2b, b2r)
    return hid[:N]
